# Initial kernel scaffold; baseline (speedup 1.0000x reference)
import jax
import jax.numpy as jnp
from jax.experimental import pallas as pl


def kernel(x, Wg, bg, W1, b1, W2, b2):
    raise NotImplementedError("write your pallas kernel here")


# speedup vs baseline: 4.1633x; 4.1633x over previous
import functools

import jax
import jax.numpy as jnp
from jax.experimental import pallas as pl
from jax.experimental.pallas import tpu as pltpu

_B, _S, _D = 2, 2048, 768
_H = 128
_E = 64
_TOPK = 8
_T = _B * _S

_TTA = 512
_TTB = 1024


def _gating_kernel(x_ref, wg_ref, bg_ref, w_ref, usage_ref, aux_ref):
    i = pl.program_id(0)
    n = pl.num_programs(0)

    s = jax.lax.dot_general(x_ref[...], wg_ref[...],
                            (((1,), (1,)), ((), ())),
                            preferred_element_type=jnp.float32)
    s = s + bg_ref[...]
    m = jnp.max(s, axis=1, keepdims=True)
    p = jnp.exp(s - m)
    probs = p / jnp.sum(p, axis=1, keepdims=True)

    @pl.when(i == 0)
    def _():
        usage_ref[...] = jnp.zeros_like(usage_ref)

    usage_ref[...] += jnp.sum(probs, axis=0, keepdims=True)

    iota = jax.lax.broadcasted_iota(jnp.int32, probs.shape, 1)
    work = probs
    sel = jnp.zeros(probs.shape, dtype=jnp.bool_)
    for _ in range(_TOPK):
        mx = jnp.max(work, axis=1, keepdims=True)
        eq = work == mx
        first_idx = jnp.min(jnp.where(eq, iota, _E), axis=1, keepdims=True)
        first = iota == first_idx
        sel = sel | first
        work = jnp.where(first, -jnp.inf, work)

    wsel = jnp.where(sel, probs, 0.0)
    w_ref[...] = wsel / jnp.sum(wsel, axis=1, keepdims=True)

    @pl.when(i == n - 1)
    def _():
        usage = usage_ref[...] / _T
        log_uniform = -jnp.log(jnp.float32(_E))
        aux = jnp.sum(usage * log_uniform - jnp.log(usage) / _E)
        aux_ref[...] = jnp.full((1, 1), aux, dtype=jnp.float32)


def _expert_kernel(x_ref, w_ref, w1_ref, b1_ref, w2_ref, b2_ref, out_ref):
    e = pl.program_id(1)

    xb = x_ref[...]
    h = jax.lax.dot_general(xb, w1_ref[0], (((1,), (1,)), ((), ())),
                            preferred_element_type=jnp.float32)
    h = h + b1_ref[0]
    h = 0.5 * h * (1.0 + jax.lax.erf(h * jnp.float32(0.7071067811865476)))
    y = jax.lax.dot_general(h, w2_ref[0], (((1,), (1,)), ((), ())),
                            preferred_element_type=jnp.float32)
    y = y + b2_ref[0]

    wall = w_ref[...]
    eiota = jax.lax.broadcasted_iota(jnp.int32, wall.shape, 1)
    wcol = jnp.sum(jnp.where(eiota == e, wall, 0.0), axis=1, keepdims=True)

    @pl.when(e == 0)
    def _():
        out_ref[...] = jnp.zeros_like(out_ref)

    out_ref[...] += wcol * y


@jax.jit
def kernel(x, Wg, bg, W1, b1, W2, b2):
    orig_shape = x.shape
    xf = x.reshape(-1, x.shape[-1])

    w, _, aux = pl.pallas_call(
        _gating_kernel,
        grid=(_T // _TTA,),
        in_specs=[
            pl.BlockSpec((_TTA, _D), lambda i: (i, 0)),
            pl.BlockSpec((_E, _D), lambda i: (0, 0)),
            pl.BlockSpec((1, _E), lambda i: (0, 0)),
        ],
        out_specs=[
            pl.BlockSpec((_TTA, _E), lambda i: (i, 0)),
            pl.BlockSpec((1, _E), lambda i: (0, 0)),
            pl.BlockSpec((1, 1), lambda i: (0, 0)),
        ],
        out_shape=[
            jax.ShapeDtypeStruct((_T, _E), jnp.float32),
            jax.ShapeDtypeStruct((1, _E), jnp.float32),
            jax.ShapeDtypeStruct((1, 1), jnp.float32),
        ],
    )(xf, Wg, bg.reshape(1, _E))

    out = pl.pallas_call(
        _expert_kernel,
        grid=(_T // _TTB, _E),
        in_specs=[
            pl.BlockSpec((_TTB, _D), lambda t, e: (t, 0)),
            pl.BlockSpec((_TTB, _E), lambda t, e: (t, 0)),
            pl.BlockSpec((1, _H, _D), lambda t, e: (e, 0, 0)),
            pl.BlockSpec((1, 1, _H), lambda t, e: (e, 0, 0)),
            pl.BlockSpec((1, _D, _H), lambda t, e: (e, 0, 0)),
            pl.BlockSpec((1, 1, _D), lambda t, e: (e, 0, 0)),
        ],
        out_specs=pl.BlockSpec((_TTB, _D), lambda t, e: (t, 0)),
        out_shape=jax.ShapeDtypeStruct((_T, _D), jnp.float32),
        compiler_params=pltpu.CompilerParams(
            dimension_semantics=("arbitrary", "arbitrary"),
        ),
    )(xf, w, W1, b1.reshape(_E, 1, _H), W2, b2.reshape(_E, 1, _D))

    return (out.reshape(orig_shape), aux[0, 0])
